# baseline (speedup 1.0000x reference)
import functools

import jax
import jax.numpy as jnp
from jax import lax
from jax.experimental import pallas as pl
from jax.experimental.pallas import tpu as pltpu
from jax.experimental.pallas import tpu_sc as plsc

f32 = jnp.float32
bf16 = jnp.bfloat16

DIM = 32
T = 32
B = 4096

NC, NS = 2, 16
NW = NC * NS

ENT_SETS = 8
REL_SETS = 4
SET = B * T
ENT_N = ENT_SETS * SET
REL_N = REL_SETS * SET
ENT_PW = ENT_N // NW
REL_PW = REL_N // NW
CH = 128
SUP = 1024
N_STREAM = SUP // CH


PK = 128 // DIM
N_ROWS = 1000000
REGION = 262144
N_PAD = PK * REGION
C4 = 2048
TGRID = REGION // C4
TAIL = N_ROWS - 3 * REGION - 212992
TAIL_BASE = N_ROWS - TAIL
TAIL_R = (TGRID - 1) * C4


def _tc_table_prep(ent_t, rel_t, ent_tail, rel_tail):
  def body(e0, e1, e2, e3, r0, r1, r2, r3, et_ref, rt_ref, eo_ref, ro_ref):
    s = pl.program_id(0)
    last = s == TGRID - 1
    zpad = jnp.zeros((C4 - TAIL, DIM), f32)
    dn = (((0,), (0,)), ((), ()))

    def emat(g):
      li = jax.lax.broadcasted_iota(jnp.int32, (DIM, 128), 1)
      di = jax.lax.broadcasted_iota(jnp.int32, (DIM, 128), 0)
      return (li == di + g * DIM).astype(f32)

    def packed(blocks, tail_ref):
      acc = None
      for g in range(PK):
        e = emat(g)
        a = jax.lax.dot_general(blocks[g][...], e, dn,
                                preferred_element_type=f32)
        if g == PK - 1:
          tb = jnp.concatenate([tail_ref[...], zpad], axis=0)
          a = jnp.where(last, jnp.dot(tb, e, preferred_element_type=f32), a)
        acc = a if acc is None else acc + a
      return acc

    eo_ref[...] = packed([e0, e1, e2, e3], et_ref).astype(bf16)
    ro_ref[...] = packed([r0, r1, r2, r3], rt_ref).astype(bf16)

  max_blk = N_ROWS // C4 - 1

  def _spec(g):
    if g == PK - 1:
      return pl.BlockSpec(
          (DIM, C4),
          lambda s, _g=g: (0, jnp.minimum(_g * TGRID + s, max_blk)))
    return pl.BlockSpec((DIM, C4), lambda s, _g=g: (0, _g * TGRID + s))

  return pl.pallas_call(
      body,
      grid=(TGRID,),
      compiler_params=pltpu.CompilerParams(fuse_transposed_lhs_in_matmul=True),
      in_specs=[_spec(g) for g in range(PK)] * 2 + [
          pl.BlockSpec((TAIL, DIM), lambda s: (0, 0)),
          pl.BlockSpec((TAIL, DIM), lambda s: (0, 0)),
      ],
      out_specs=[
          pl.BlockSpec((C4, 128), lambda s: (s, 0)),
          pl.BlockSpec((C4, 128), lambda s: (s, 0)),
      ],
      out_shape=[
          jax.ShapeDtypeStruct((REGION, 128), bf16),
          jax.ShapeDtypeStruct((REGION, 128), bf16),
      ],
  )(ent_t, ent_t, ent_t, ent_t, rel_t, rel_t, rel_t, rel_t,
    ent_tail, rel_tail)


def _sc_gather(ent_emb, rel_emb, ent_idx, rel_idx, items):
  mesh = plsc.VectorSubcoreMesh(core_axis_name="c", subcore_axis_name="s",
                                num_cores=NC, num_subcores=NS)

  @functools.partial(
      pl.kernel,
      out_type=(
          jax.ShapeDtypeStruct((ENT_N, DIM), bf16),
          jax.ShapeDtypeStruct((REL_N, DIM), bf16),
          jax.ShapeDtypeStruct((B, DIM), bf16),
      ),
      mesh=mesh,
      compiler_params=pltpu.CompilerParams(use_tc_tiling_on_sc=False),
      scratch_types=[
          pltpu.VMEM((SUP,), jnp.int32),
          pltpu.VMEM((SUP, DIM), bf16),
          pltpu.SemaphoreType.DMA,
      ],
  )
  def gather_kernel(ent_hbm, rel_hbm, eidx_hbm, ridx_hbm, item_hbm,
                    eout_hbm, rout_hbm, iout_hbm, idx_v, rows_v, sem):
    wid = lax.axis_index("s") * NC + lax.axis_index("c")

    def table_loop(tab_hbm, idx_hbm, out_hbm, base, n_sup):
      def body(i, carry):
        off = pl.multiple_of(base + i * SUP, SUP)
        pltpu.sync_copy(idx_hbm.at[pl.ds(off, SUP)], idx_v)
        cps = [
            pltpu.async_copy(
                tab_hbm.at[idx_v.at[pl.ds(j * CH, CH)]],
                rows_v.at[pl.ds(j * CH, CH)], sem)
            for j in range(N_STREAM)
        ]
        for cp in cps:
          cp.wait()
        pltpu.sync_copy(rows_v, out_hbm.at[pl.ds(off, SUP)])
        return carry
      lax.fori_loop(0, n_sup, body, 0)

    table_loop(ent_hbm, eidx_hbm, eout_hbm, wid * ENT_PW, ENT_PW // SUP)
    table_loop(rel_hbm, ridx_hbm, rout_hbm, wid * REL_PW, REL_PW // SUP)
    ioff = wid * (B // NW)
    pltpu.sync_copy(item_hbm.at[pl.ds(ioff, CH)], idx_v.at[pl.ds(0, CH)])
    pltpu.async_copy(ent_hbm.at[idx_v.at[pl.ds(0, CH)]],
                     rows_v.at[pl.ds(0, CH)], sem).wait()
    pltpu.sync_copy(rows_v.at[pl.ds(0, CH)], iout_hbm.at[pl.ds(ioff, CH)])

  return gather_kernel(ent_emb, rel_emb, ent_idx, rel_idx, items)


BB = 256
GRID = B // BB
PR = T // PK
BBR = BB * PR
SET_P = SET // PK


def _pack4(m):
  return jnp.concatenate([m, m, m, m], axis=0)


def _tile4(m):
  return jnp.concatenate([m, m, m, m], axis=1)


def _tc_body(ent_ref, rel_ref, item_ref, w1_ref, w2_ref, w3_ref, out_ref):
  gi = jax.lax.broadcasted_iota(jnp.int32, (128, 128), 0) // DIM
  gj = jax.lax.broadcasted_iota(jnp.int32, (128, 128), 1) // DIM
  gmask = (gi == gj).astype(f32)
  w1ap = (_tile4(_pack4(w1_ref[0:DIM, :])) * gmask).astype(bf16)
  w1bp = (_tile4(_pack4(w1_ref[DIM:2 * DIM, :])) * gmask).astype(bf16)
  w2p = (_tile4(_pack4(w2_ref[...])) * gmask).astype(bf16)
  m3 = (jnp.broadcast_to(_pack4(w3_ref[...]), (128, 128)) * gmask).astype(bf16)

  def attend(hp, rp, tp):
    x = jax.nn.relu(jnp.dot(hp, w1ap, preferred_element_type=f32)
                    + jnp.dot(rp, w1bp, preferred_element_type=f32))
    x = jax.nn.relu(jnp.dot(x.astype(bf16), w2p, preferred_element_type=f32))
    sb = jnp.dot(x.astype(bf16), m3, preferred_element_type=f32)
    pe = jnp.exp(jax.nn.sigmoid(sb)).reshape(BB, PR, 128)
    denom = pe.sum(axis=1).sum(axis=-1, keepdims=True) * (1.0 / T)
    att = pe / denom[:, :, None]
    tpf = tp.astype(f32).reshape(BB, PR, 128)
    return (att * tpf).sum(axis=1)

  u0p = attend(ent_ref[0], rel_ref[0], ent_ref[1])
  u1p = attend(ent_ref[2], rel_ref[1], ent_ref[3])
  i0p = attend(ent_ref[4], rel_ref[2], ent_ref[5])
  i1p = attend(ent_ref[6], rel_ref[3], ent_ref[7])
  uorg_p = ent_ref[0].astype(f32).reshape(BB, PR, 128).sum(axis=1)
  iorg4 = _tile4(item_ref[...])

  fl = jax.lax.broadcasted_iota(jnp.int32, (128, DIM), 0) % DIM
  fd = jax.lax.broadcasted_iota(jnp.int32, (128, DIM), 1)
  fmat = (fl == fd).astype(f32)
  vl0 = _tile4(jnp.dot(i0p, fmat, preferred_element_type=f32))
  vl1 = _tile4(jnp.dot(i1p, fmat, preferred_element_type=f32))

  pre = ((uorg_p * iorg4).sum(axis=-1) * (1.0 / T)
         + (u0p * vl0).sum(axis=-1) + (u1p * vl1).sum(axis=-1))
  out_ref[...] = jax.nn.sigmoid(pre)


def _tc_dense(ent3, rel3, item_rows, W1, W2, W3):
  return pl.pallas_call(
      _tc_body,
      grid=(GRID,),
      in_specs=[
          pl.BlockSpec((ENT_SETS, BBR, 128), lambda g: (0, g, 0)),
          pl.BlockSpec((REL_SETS, BBR, 128), lambda g: (0, g, 0)),
          pl.BlockSpec((BB, DIM), lambda g: (g, 0)),
          pl.BlockSpec((2 * DIM, DIM), lambda g: (0, 0)),
          pl.BlockSpec((DIM, DIM), lambda g: (0, 0)),
          pl.BlockSpec((DIM, 1), lambda g: (0, 0)),
      ],
      out_specs=pl.BlockSpec((BB,), lambda g: (g,)),
      out_shape=jax.ShapeDtypeStruct((B,), f32),
  )(ent3, rel3, item_rows, W1, W2, W3)


def kernel(users, items, user_triple_set, item_triple_set,
           entity_emb, relation_emb, W1, W2, W3):
  uts = user_triple_set
  its = item_triple_set
  ent_idx = jnp.concatenate([
      uts[0, 0].reshape(-1), uts[2, 0].reshape(-1),
      uts[0, 1].reshape(-1), uts[2, 1].reshape(-1),
      its[0, 0].reshape(-1), its[2, 0].reshape(-1),
      its[0, 1].reshape(-1), its[2, 1].reshape(-1),
  ])
  rel_idx = jnp.concatenate([
      uts[1, 0].reshape(-1), uts[1, 1].reshape(-1),
      its[1, 0].reshape(-1), its[1, 1].reshape(-1),
  ])
  def _remap(i):
    return jnp.where(i < TAIL_BASE,
                     (i % REGION) * PK + i // REGION,
                     (i - TAIL_BASE + TAIL_R) * PK + (PK - 1))

  ent_idx = _remap(ent_idx)
  rel_idx = _remap(rel_idx)
  item_idx = _remap(items)
  ent_lin, rel_lin = _tc_table_prep(
      entity_emb.T, relation_emb.T,
      entity_emb[TAIL_BASE:], relation_emb[TAIL_BASE:])
  ent_rows, rel_rows, item_rows = _sc_gather(
      ent_lin.reshape(N_PAD, DIM), rel_lin.reshape(N_PAD, DIM),
      ent_idx, rel_idx, item_idx)
  ent3 = ent_rows.reshape(ENT_SETS, SET_P, 128)
  rel3 = rel_rows.reshape(REL_SETS, SET_P, 128)
  scores = _tc_dense(ent3, rel3, item_rows, W1, W2, W3)
  return scores, 0.0, 0.0, 0.0


# speedup vs baseline: 2.2453x; 2.2453x over previous
import functools

import jax
import jax.numpy as jnp
from jax import lax
from jax.experimental import pallas as pl
from jax.experimental.pallas import tpu as pltpu
from jax.experimental.pallas import tpu_sc as plsc

f32 = jnp.float32

DIM = 32
T = 32
B = 4096

NC, NS = 2, 16
NW = NC * NS

ENT_SETS = 8
REL_SETS = 4
SET = B * T
ENT_N = ENT_SETS * SET
REL_N = REL_SETS * SET
ENT_PW = ENT_N // NW
REL_PW = REL_N // NW
CH = 128
SUP = 1024
N_STREAM = SUP // CH


PK = 128 // DIM
N_ROWS = 1000000
REGION = 262144
N_PAD = PK * REGION
C4 = 2048
TGRID = REGION // C4
TAIL = N_ROWS - 3 * REGION - 212992
TAIL_BASE = N_ROWS - TAIL
TAIL_R = (TGRID - 1) * C4


def _tc_table_prep(tab_t, tab_tail):
  def body(e0, e1, e2, e3, tail_ref, out_ref):
    s = pl.program_id(0)
    last = s == TGRID - 1
    zpad = jnp.zeros((C4 - TAIL, DIM), f32)
    dn = (((0,), (0,)), ((), ()))

    def emat(g):
      li = jax.lax.broadcasted_iota(jnp.int32, (DIM, 128), 1)
      di = jax.lax.broadcasted_iota(jnp.int32, (DIM, 128), 0)
      return (li == di + g * DIM).astype(f32)

    blocks = [e0, e1, e2, e3]
    acc = None
    for g in range(PK):
      e = emat(g)
      a = jax.lax.dot_general(blocks[g][...], e, dn,
                              preferred_element_type=f32)
      if g == PK - 1:
        tb = jnp.concatenate([tail_ref[...], zpad], axis=0)
        a = jnp.where(last, jnp.dot(tb, e, preferred_element_type=f32), a)
      acc = a if acc is None else acc + a
    out_ref[...] = acc

  max_blk = N_ROWS // C4 - 1

  def _spec(g):
    if g == PK - 1:
      return pl.BlockSpec(
          (DIM, C4),
          lambda s, _g=g: (0, jnp.minimum(_g * TGRID + s, max_blk)))
    return pl.BlockSpec((DIM, C4), lambda s, _g=g: (0, _g * TGRID + s))

  return pl.pallas_call(
      body,
      grid=(TGRID,),
      compiler_params=pltpu.CompilerParams(fuse_transposed_lhs_in_matmul=True),
      in_specs=[_spec(g) for g in range(PK)] + [
          pl.BlockSpec((TAIL, DIM), lambda s: (0, 0)),
      ],
      out_specs=pl.BlockSpec((C4, 128), lambda s: (s, 0)),
      out_shape=jax.ShapeDtypeStruct((REGION, 128), f32),
  )(tab_t, tab_t, tab_t, tab_t, tab_tail)


def _sc_mesh():
  return plsc.VectorSubcoreMesh(core_axis_name="c", subcore_axis_name="s",
                                num_cores=NC, num_subcores=NS)


def _sc_table_loop(tab_hbm, idx_hbm, out_hbm, idx_v, rows_v, sem, base, n_sup):
  def body(i, carry):
    off = pl.multiple_of(base + i * SUP, SUP)
    pltpu.sync_copy(idx_hbm.at[pl.ds(off, SUP)], idx_v)
    cps = [
        pltpu.async_copy(
            tab_hbm.at[idx_v.at[pl.ds(j * CH, CH)]],
            rows_v.at[pl.ds(j * CH, CH)], sem)
        for j in range(N_STREAM)
    ]
    for cp in cps:
      cp.wait()
    pltpu.sync_copy(rows_v, out_hbm.at[pl.ds(off, SUP)])
    return carry
  lax.fori_loop(0, n_sup, body, 0)


_SC_SCRATCH = [
    pltpu.VMEM((SUP,), jnp.int32),
    pltpu.VMEM((SUP, DIM), f32),
    pltpu.SemaphoreType.DMA,
]


def _sc_gather_ent(ent_emb, ent_idx, items):
  @functools.partial(
      pl.kernel,
      out_type=(
          jax.ShapeDtypeStruct((ENT_N, DIM), f32),
          jax.ShapeDtypeStruct((B, DIM), f32),
      ),
      mesh=_sc_mesh(),
      compiler_params=pltpu.CompilerParams(use_tc_tiling_on_sc=False),
      scratch_types=list(_SC_SCRATCH),
  )
  def gather_kernel(ent_hbm, eidx_hbm, item_hbm, eout_hbm, iout_hbm,
                    idx_v, rows_v, sem):
    wid = lax.axis_index("s") * NC + lax.axis_index("c")
    _sc_table_loop(ent_hbm, eidx_hbm, eout_hbm, idx_v, rows_v, sem,
                   wid * ENT_PW, ENT_PW // SUP)
    ioff = wid * (B // NW)
    pltpu.sync_copy(item_hbm.at[pl.ds(ioff, CH)], idx_v.at[pl.ds(0, CH)])
    pltpu.async_copy(ent_hbm.at[idx_v.at[pl.ds(0, CH)]],
                     rows_v.at[pl.ds(0, CH)], sem).wait()
    pltpu.sync_copy(rows_v.at[pl.ds(0, CH)], iout_hbm.at[pl.ds(ioff, CH)])

  return gather_kernel(ent_emb, ent_idx, items)


def _sc_gather_rel(rel_emb, rel_idx):
  @functools.partial(
      pl.kernel,
      out_type=jax.ShapeDtypeStruct((REL_N, DIM), f32),
      mesh=_sc_mesh(),
      compiler_params=pltpu.CompilerParams(use_tc_tiling_on_sc=False),
      scratch_types=list(_SC_SCRATCH),
  )
  def gather_kernel(rel_hbm, ridx_hbm, rout_hbm, idx_v, rows_v, sem):
    wid = lax.axis_index("s") * NC + lax.axis_index("c")
    _sc_table_loop(rel_hbm, ridx_hbm, rout_hbm, idx_v, rows_v, sem,
                   wid * REL_PW, REL_PW // SUP)

  return gather_kernel(rel_emb, rel_idx)


BB = 256
GRID = B // BB
PR = T // PK
BBR = BB * PR
SET_P = SET // PK


def _pack4(m):
  return jnp.concatenate([m, m, m, m], axis=0)


def _tile4(m):
  return jnp.concatenate([m, m, m, m], axis=1)


def _tc_body(ent_ref, rel_ref, item_ref, w1_ref, w2_ref, w3_ref, out_ref):
  gi = jax.lax.broadcasted_iota(jnp.int32, (128, 128), 0) // DIM
  gj = jax.lax.broadcasted_iota(jnp.int32, (128, 128), 1) // DIM
  gmask = (gi == gj).astype(f32)
  w1ap = _tile4(_pack4(w1_ref[0:DIM, :])) * gmask
  w1bp = _tile4(_pack4(w1_ref[DIM:2 * DIM, :])) * gmask
  w2p = _tile4(_pack4(w2_ref[...])) * gmask
  m3 = jnp.broadcast_to(_pack4(w3_ref[...]), (128, 128)) * gmask

  def attend(hp, rp, tp):
    x = jax.nn.relu(jnp.dot(hp, w1ap, preferred_element_type=f32)
                    + jnp.dot(rp, w1bp, preferred_element_type=f32))
    x = jax.nn.relu(jnp.dot(x, w2p, preferred_element_type=f32))
    sb = jnp.dot(x, m3, preferred_element_type=f32)
    pe = jnp.exp(jax.nn.sigmoid(sb)).reshape(BB, PR, 128)
    denom = pe.sum(axis=1).sum(axis=-1, keepdims=True) * (1.0 / T)
    att = pe / denom[:, :, None]
    tpf = tp.astype(f32).reshape(BB, PR, 128)
    return (att * tpf).sum(axis=1)

  u0p = attend(ent_ref[0], rel_ref[0], ent_ref[1])
  u1p = attend(ent_ref[2], rel_ref[1], ent_ref[3])
  i0p = attend(ent_ref[4], rel_ref[2], ent_ref[5])
  i1p = attend(ent_ref[6], rel_ref[3], ent_ref[7])
  uorg_p = ent_ref[0].astype(f32).reshape(BB, PR, 128).sum(axis=1)
  iorg4 = _tile4(item_ref[...])

  fl = jax.lax.broadcasted_iota(jnp.int32, (128, DIM), 0) % DIM
  fd = jax.lax.broadcasted_iota(jnp.int32, (128, DIM), 1)
  fmat = (fl == fd).astype(f32)
  vl0 = _tile4(jnp.dot(i0p, fmat, preferred_element_type=f32))
  vl1 = _tile4(jnp.dot(i1p, fmat, preferred_element_type=f32))

  pre = ((uorg_p * iorg4).sum(axis=-1) * (1.0 / T)
         + (u0p * vl0).sum(axis=-1) + (u1p * vl1).sum(axis=-1))
  out_ref[...] = jax.nn.sigmoid(pre)


def _tc_dense(ent3, rel3, item_rows, W1, W2, W3):
  return pl.pallas_call(
      _tc_body,
      grid=(GRID,),
      in_specs=[
          pl.BlockSpec((ENT_SETS, BBR, 128), lambda g: (0, g, 0)),
          pl.BlockSpec((REL_SETS, BBR, 128), lambda g: (0, g, 0)),
          pl.BlockSpec((BB, DIM), lambda g: (g, 0)),
          pl.BlockSpec((2 * DIM, DIM), lambda g: (0, 0)),
          pl.BlockSpec((DIM, DIM), lambda g: (0, 0)),
          pl.BlockSpec((DIM, 1), lambda g: (0, 0)),
      ],
      out_specs=pl.BlockSpec((BB,), lambda g: (g,)),
      out_shape=jax.ShapeDtypeStruct((B,), f32),
  )(ent3, rel3, item_rows, W1, W2, W3)


def kernel(users, items, user_triple_set, item_triple_set,
           entity_emb, relation_emb, W1, W2, W3):
  uts = user_triple_set
  its = item_triple_set
  ent_idx = jnp.concatenate([
      uts[0, 0].reshape(-1), uts[2, 0].reshape(-1),
      uts[0, 1].reshape(-1), uts[2, 1].reshape(-1),
      its[0, 0].reshape(-1), its[2, 0].reshape(-1),
      its[0, 1].reshape(-1), its[2, 1].reshape(-1),
  ])
  rel_idx = jnp.concatenate([
      uts[1, 0].reshape(-1), uts[1, 1].reshape(-1),
      its[1, 0].reshape(-1), its[1, 1].reshape(-1),
  ])
  def _remap(i):
    return jnp.where(i < TAIL_BASE,
                     (i % REGION) * PK + i // REGION,
                     (i - TAIL_BASE + TAIL_R) * PK + (PK - 1))

  ent_idx = _remap(ent_idx)
  rel_idx = _remap(rel_idx)
  item_idx = _remap(items)
  ent_lin = _tc_table_prep(entity_emb.T, entity_emb[TAIL_BASE:])
  ent_rows, item_rows = _sc_gather_ent(
      ent_lin.reshape(N_PAD, DIM), ent_idx, item_idx)
  rel_lin = _tc_table_prep(relation_emb.T, relation_emb[TAIL_BASE:])
  rel_rows = _sc_gather_rel(rel_lin.reshape(N_PAD, DIM), rel_idx)
  ent3 = ent_rows.reshape(ENT_SETS, SET_P, 128)
  rel3 = rel_rows.reshape(REL_SETS, SET_P, 128)
  scores = _tc_dense(ent3, rel3, item_rows, W1, W2, W3)
  return scores, 0.0, 0.0, 0.0


# speedup vs baseline: 2.6067x; 1.1609x over previous
import functools

import jax
import jax.numpy as jnp
from jax import lax
from jax.experimental import pallas as pl
from jax.experimental.pallas import tpu as pltpu
from jax.experimental.pallas import tpu_sc as plsc

f32 = jnp.float32

DIM = 32
T = 32
B = 4096

NC, NS = 2, 16
NW = NC * NS

ENT_SETS = 8
REL_SETS = 4
SET = B * T
ENT_N = ENT_SETS * SET
REL_N = REL_SETS * SET
ENT_PW = ENT_N // NW
REL_PW = REL_N // NW
CH = 128
SUP = 1024
N_STREAM = SUP // CH


PK = 128 // DIM
N_ROWS = 1000000
REGION = 262144
N_PAD = PK * REGION
C4 = 4096
TGRID = REGION // C4
TAIL = N_ROWS - 3 * REGION - 212992
TAIL_BASE = N_ROWS - TAIL
TAIL_R = (TGRID - 1) * C4


def _tc_table_prep(tab_t, tab_tail):
  def body(e0, e1, e2, e3, tail_ref, out_ref):
    s = pl.program_id(0)
    last = s == TGRID - 1
    zpad = jnp.zeros((C4 - TAIL, DIM), f32)
    dn = (((0,), (0,)), ((), ()))

    def emat(g):
      li = jax.lax.broadcasted_iota(jnp.int32, (DIM, 128), 1)
      di = jax.lax.broadcasted_iota(jnp.int32, (DIM, 128), 0)
      return (li == di + g * DIM).astype(f32)

    blocks = [e0, e1, e2, e3]
    acc = None
    for g in range(PK):
      e = emat(g)
      a = jax.lax.dot_general(blocks[g][...], e, dn,
                              preferred_element_type=f32)
      if g == PK - 1:
        tb = jnp.concatenate([tail_ref[...], zpad], axis=0)
        a = jnp.where(last, jnp.dot(tb, e, preferred_element_type=f32), a)
      acc = a if acc is None else acc + a
    out_ref[...] = acc

  max_blk = N_ROWS // C4 - 1

  def _spec(g):
    if g == PK - 1:
      return pl.BlockSpec(
          (DIM, C4),
          lambda s, _g=g: (0, jnp.minimum(_g * TGRID + s, max_blk)))
    return pl.BlockSpec((DIM, C4), lambda s, _g=g: (0, _g * TGRID + s))

  return pl.pallas_call(
      body,
      grid=(TGRID,),
      compiler_params=pltpu.CompilerParams(fuse_transposed_lhs_in_matmul=True),
      in_specs=[_spec(g) for g in range(PK)] + [
          pl.BlockSpec((TAIL, DIM), lambda s: (0, 0)),
      ],
      out_specs=pl.BlockSpec((C4, 128), lambda s: (s, 0)),
      out_shape=jax.ShapeDtypeStruct((REGION, 128), f32),
  )(tab_t, tab_t, tab_t, tab_t, tab_tail)


def _sc_mesh():
  return plsc.VectorSubcoreMesh(core_axis_name="c", subcore_axis_name="s",
                                num_cores=NC, num_subcores=NS)


def _sc_table_loop(tab_hbm, idx_hbm, out_hbm, idx_v, rows_v, sem, base, n_sup):
  def body(i, carry):
    off = pl.multiple_of(base + i * SUP, SUP)
    pltpu.sync_copy(idx_hbm.at[pl.ds(off, SUP)], idx_v)
    cps = [
        pltpu.async_copy(
            tab_hbm.at[idx_v.at[pl.ds(j * CH, CH)]],
            rows_v.at[pl.ds(j * CH, CH)], sem)
        for j in range(N_STREAM)
    ]
    for cp in cps:
      cp.wait()
    pltpu.sync_copy(rows_v, out_hbm.at[pl.ds(off, SUP)])
    return carry
  lax.fori_loop(0, n_sup, body, 0)


_SC_SCRATCH = [
    pltpu.VMEM((SUP,), jnp.int32),
    pltpu.VMEM((SUP, DIM), f32),
    pltpu.SemaphoreType.DMA,
]


def _sc_gather_ent(ent_emb, ent_idx, items):
  @functools.partial(
      pl.kernel,
      out_type=(
          jax.ShapeDtypeStruct((ENT_N, DIM), f32),
          jax.ShapeDtypeStruct((B, DIM), f32),
      ),
      mesh=_sc_mesh(),
      compiler_params=pltpu.CompilerParams(use_tc_tiling_on_sc=False),
      scratch_types=list(_SC_SCRATCH),
  )
  def gather_kernel(ent_hbm, eidx_hbm, item_hbm, eout_hbm, iout_hbm,
                    idx_v, rows_v, sem):
    wid = lax.axis_index("s") * NC + lax.axis_index("c")
    _sc_table_loop(ent_hbm, eidx_hbm, eout_hbm, idx_v, rows_v, sem,
                   wid * ENT_PW, ENT_PW // SUP)
    ioff = wid * (B // NW)
    pltpu.sync_copy(item_hbm.at[pl.ds(ioff, CH)], idx_v.at[pl.ds(0, CH)])
    pltpu.async_copy(ent_hbm.at[idx_v.at[pl.ds(0, CH)]],
                     rows_v.at[pl.ds(0, CH)], sem).wait()
    pltpu.sync_copy(rows_v.at[pl.ds(0, CH)], iout_hbm.at[pl.ds(ioff, CH)])

  return gather_kernel(ent_emb, ent_idx, items)


REL_NH = REL_N // 2
REL_PWH = REL_NH // NW


def _sc_gather_rel(rel_emb, rel_idx_h):
  @functools.partial(
      pl.kernel,
      out_type=jax.ShapeDtypeStruct((REL_NH, DIM), f32),
      mesh=_sc_mesh(),
      compiler_params=pltpu.CompilerParams(use_tc_tiling_on_sc=False),
      scratch_types=list(_SC_SCRATCH),
  )
  def gather_kernel(rel_hbm, ridx_hbm, rout_hbm, idx_v, rows_v, sem):
    wid = lax.axis_index("s") * NC + lax.axis_index("c")
    _sc_table_loop(rel_hbm, ridx_hbm, rout_hbm, idx_v, rows_v, sem,
                   wid * REL_PWH, REL_PWH // SUP)

  return gather_kernel(rel_emb, rel_idx_h)


BB = 256
GRID = B // BB
PR = T // PK
BBR = BB * PR
SET_P = SET // PK


def _pack4(m):
  return jnp.concatenate([m, m, m, m], axis=0)


def _tile4(m):
  return jnp.concatenate([m, m, m, m], axis=1)


def _tc_body(ent_ref, rel_ref, item_ref, w1_ref, w2_ref, w3_ref, out_ref):
  gi = jax.lax.broadcasted_iota(jnp.int32, (128, 128), 0) // DIM
  gj = jax.lax.broadcasted_iota(jnp.int32, (128, 128), 1) // DIM
  gmask = (gi == gj).astype(f32)
  w1ap = _tile4(_pack4(w1_ref[0:DIM, :])) * gmask
  w1bp = _tile4(_pack4(w1_ref[DIM:2 * DIM, :])) * gmask
  w2p = _tile4(_pack4(w2_ref[...])) * gmask
  m3 = jnp.broadcast_to(_pack4(w3_ref[...]), (128, 128)) * gmask

  def attend(hp, rp, tp):
    x = jax.nn.relu(jnp.dot(hp, w1ap, preferred_element_type=f32)
                    + jnp.dot(rp, w1bp, preferred_element_type=f32))
    x = jax.nn.relu(jnp.dot(x, w2p, preferred_element_type=f32))
    sb = jnp.dot(x, m3, preferred_element_type=f32)
    pe = jnp.exp(jax.nn.sigmoid(sb)).reshape(BB, PR, 128)
    denom = pe.sum(axis=1).sum(axis=-1, keepdims=True) * (1.0 / T)
    att = pe / denom[:, :, None]
    tpf = tp.astype(f32).reshape(BB, PR, 128)
    return (att * tpf).sum(axis=1)

  u0p = attend(ent_ref[0], rel_ref[0], ent_ref[1])
  u1p = attend(ent_ref[2], rel_ref[1], ent_ref[3])
  i0p = attend(ent_ref[4], rel_ref[2], ent_ref[5])
  i1p = attend(ent_ref[6], rel_ref[3], ent_ref[7])
  uorg_p = ent_ref[0].astype(f32).reshape(BB, PR, 128).sum(axis=1)
  iorg4 = _tile4(item_ref[...])

  fl = jax.lax.broadcasted_iota(jnp.int32, (128, DIM), 0) % DIM
  fd = jax.lax.broadcasted_iota(jnp.int32, (128, DIM), 1)
  fmat = (fl == fd).astype(f32)
  vl0 = _tile4(jnp.dot(i0p, fmat, preferred_element_type=f32))
  vl1 = _tile4(jnp.dot(i1p, fmat, preferred_element_type=f32))

  pre = ((uorg_p * iorg4).sum(axis=-1) * (1.0 / T)
         + (u0p * vl0).sum(axis=-1) + (u1p * vl1).sum(axis=-1))
  out_ref[...] = jax.nn.sigmoid(pre)


def _tc_dense(ent3, rel3h, item_rows, W1, W2, W3, goff):
  return pl.pallas_call(
      _tc_body,
      grid=(GRID // 2,),
      in_specs=[
          pl.BlockSpec((ENT_SETS, BBR, 128), lambda g: (0, g + goff, 0)),
          pl.BlockSpec((REL_SETS, BBR, 128), lambda g: (0, g, 0)),
          pl.BlockSpec((BB, DIM), lambda g: (g + goff, 0)),
          pl.BlockSpec((2 * DIM, DIM), lambda g: (0, 0)),
          pl.BlockSpec((DIM, DIM), lambda g: (0, 0)),
          pl.BlockSpec((DIM, 1), lambda g: (0, 0)),
      ],
      out_specs=pl.BlockSpec((BB,), lambda g: (g,)),
      out_shape=jax.ShapeDtypeStruct((B // 2,), f32),
  )(ent3, rel3h, item_rows, W1, W2, W3)


def kernel(users, items, user_triple_set, item_triple_set,
           entity_emb, relation_emb, W1, W2, W3):
  uts = user_triple_set
  its = item_triple_set
  ent_idx = jnp.concatenate([
      uts[0, 0].reshape(-1), uts[2, 0].reshape(-1),
      uts[0, 1].reshape(-1), uts[2, 1].reshape(-1),
      its[0, 0].reshape(-1), its[2, 0].reshape(-1),
      its[0, 1].reshape(-1), its[2, 1].reshape(-1),
  ])
  hb = B // 2
  rel_idx_h = [jnp.concatenate([
      uts[1, 0, h * hb:(h + 1) * hb].reshape(-1),
      uts[1, 1, h * hb:(h + 1) * hb].reshape(-1),
      its[1, 0, h * hb:(h + 1) * hb].reshape(-1),
      its[1, 1, h * hb:(h + 1) * hb].reshape(-1),
  ]) for h in range(2)]
  def _remap(i):
    return jnp.where(i < TAIL_BASE,
                     (i % REGION) * PK + i // REGION,
                     (i - TAIL_BASE + TAIL_R) * PK + (PK - 1))

  ent_idx = _remap(ent_idx)
  rel_idx_h = [_remap(r) for r in rel_idx_h]
  item_idx = _remap(items)
  ent_lin = _tc_table_prep(entity_emb.T, entity_emb[TAIL_BASE:])
  ent_rows, item_rows = _sc_gather_ent(
      ent_lin.reshape(N_PAD, DIM), ent_idx, item_idx)
  rel_lin = _tc_table_prep(relation_emb.T, relation_emb[TAIL_BASE:])
  rel_h = [_sc_gather_rel(rel_lin.reshape(N_PAD, DIM), r)
           for r in rel_idx_h]
  ent3 = ent_rows.reshape(ENT_SETS, SET_P, 128)
  halves = [
      _tc_dense(ent3, rel_h[h].reshape(REL_SETS, SET_P // 2, 128),
                item_rows, W1, W2, W3, h * (GRID // 2))
      for h in range(2)
  ]
  scores = jnp.concatenate(halves)
  return scores, 0.0, 0.0, 0.0


# speedup vs baseline: 2.6488x; 1.0162x over previous
import functools

import jax
import jax.numpy as jnp
from jax import lax
from jax.experimental import pallas as pl
from jax.experimental.pallas import tpu as pltpu
from jax.experimental.pallas import tpu_sc as plsc

f32 = jnp.float32

DIM = 32
T = 32
B = 4096

NC, NS = 2, 16
NW = NC * NS

ENT_SETS = 8
REL_SETS = 4
SET = B * T
ENT_N = ENT_SETS * SET
REL_N = REL_SETS * SET
ENT_PW = ENT_N // NW
REL_PW = REL_N // NW
CH = 128
SUP = 1024
N_STREAM = SUP // CH


PK = 128 // DIM
N_ROWS = 1000000
REGION = 262144
N_PAD = PK * REGION
C4 = 8192
TGRID = REGION // C4
TAIL = N_ROWS - 3 * REGION - 212992
TAIL_BASE = N_ROWS - TAIL
TAIL_R = (TGRID - 1) * C4


def _tc_table_prep(tab_t, tab_tail):
  def body(e0, e1, e2, e3, tail_ref, out_ref):
    s = pl.program_id(0)
    last = s == TGRID - 1
    zpad = jnp.zeros((C4 - TAIL, DIM), f32)
    dn = (((0,), (0,)), ((), ()))

    def emat(g):
      li = jax.lax.broadcasted_iota(jnp.int32, (DIM, 128), 1)
      di = jax.lax.broadcasted_iota(jnp.int32, (DIM, 128), 0)
      return (li == di + g * DIM).astype(f32)

    blocks = [e0, e1, e2, e3]
    acc = None
    for g in range(PK):
      e = emat(g)
      a = jax.lax.dot_general(blocks[g][...], e, dn,
                              preferred_element_type=f32)
      if g == PK - 1:
        tb = jnp.concatenate([tail_ref[...], zpad], axis=0)
        a = jnp.where(last, jnp.dot(tb, e, preferred_element_type=f32), a)
      acc = a if acc is None else acc + a
    out_ref[...] = acc

  max_blk = N_ROWS // C4 - 1

  def _spec(g):
    if g == PK - 1:
      return pl.BlockSpec(
          (DIM, C4),
          lambda s, _g=g: (0, jnp.minimum(_g * TGRID + s, max_blk)))
    return pl.BlockSpec((DIM, C4), lambda s, _g=g: (0, _g * TGRID + s))

  return pl.pallas_call(
      body,
      grid=(TGRID,),
      compiler_params=pltpu.CompilerParams(fuse_transposed_lhs_in_matmul=True),
      in_specs=[_spec(g) for g in range(PK)] + [
          pl.BlockSpec((TAIL, DIM), lambda s: (0, 0)),
      ],
      out_specs=pl.BlockSpec((C4, 128), lambda s: (s, 0)),
      out_shape=jax.ShapeDtypeStruct((REGION, 128), f32),
  )(tab_t, tab_t, tab_t, tab_t, tab_tail)


def _sc_mesh():
  return plsc.VectorSubcoreMesh(core_axis_name="c", subcore_axis_name="s",
                                num_cores=NC, num_subcores=NS)


def _sc_table_loop(tab_hbm, idx_hbm, out_hbm, idx_v, rows_v, sem, base, n_sup):
  def body(i, carry):
    off = pl.multiple_of(base + i * SUP, SUP)
    pltpu.sync_copy(idx_hbm.at[pl.ds(off, SUP)], idx_v)
    cps = [
        pltpu.async_copy(
            tab_hbm.at[idx_v.at[pl.ds(j * CH, CH)]],
            rows_v.at[pl.ds(j * CH, CH)], sem)
        for j in range(N_STREAM)
    ]
    for cp in cps:
      cp.wait()
    pltpu.sync_copy(rows_v, out_hbm.at[pl.ds(off, SUP)])
    return carry
  lax.fori_loop(0, n_sup, body, 0)


_SC_SCRATCH = [
    pltpu.VMEM((SUP,), jnp.int32),
    pltpu.VMEM((SUP, DIM), f32),
    pltpu.SemaphoreType.DMA,
]


def _sc_gather_ent(ent_emb, ent_idx, items):
  @functools.partial(
      pl.kernel,
      out_type=(
          jax.ShapeDtypeStruct((ENT_N, DIM), f32),
          jax.ShapeDtypeStruct((B, DIM), f32),
      ),
      mesh=_sc_mesh(),
      compiler_params=pltpu.CompilerParams(use_tc_tiling_on_sc=False),
      scratch_types=list(_SC_SCRATCH),
  )
  def gather_kernel(ent_hbm, eidx_hbm, item_hbm, eout_hbm, iout_hbm,
                    idx_v, rows_v, sem):
    wid = lax.axis_index("s") * NC + lax.axis_index("c")
    _sc_table_loop(ent_hbm, eidx_hbm, eout_hbm, idx_v, rows_v, sem,
                   wid * ENT_PW, ENT_PW // SUP)
    ioff = wid * (B // NW)
    pltpu.sync_copy(item_hbm.at[pl.ds(ioff, CH)], idx_v.at[pl.ds(0, CH)])
    pltpu.async_copy(ent_hbm.at[idx_v.at[pl.ds(0, CH)]],
                     rows_v.at[pl.ds(0, CH)], sem).wait()
    pltpu.sync_copy(rows_v.at[pl.ds(0, CH)], iout_hbm.at[pl.ds(ioff, CH)])

  return gather_kernel(ent_emb, ent_idx, items)


REL_NH = REL_N // 2
REL_PWH = REL_NH // NW


def _sc_gather_rel(rel_emb, rel_idx_h):
  @functools.partial(
      pl.kernel,
      out_type=jax.ShapeDtypeStruct((REL_NH, DIM), f32),
      mesh=_sc_mesh(),
      compiler_params=pltpu.CompilerParams(use_tc_tiling_on_sc=False),
      scratch_types=list(_SC_SCRATCH),
  )
  def gather_kernel(rel_hbm, ridx_hbm, rout_hbm, idx_v, rows_v, sem):
    wid = lax.axis_index("s") * NC + lax.axis_index("c")
    _sc_table_loop(rel_hbm, ridx_hbm, rout_hbm, idx_v, rows_v, sem,
                   wid * REL_PWH, REL_PWH // SUP)

  return gather_kernel(rel_emb, rel_idx_h)


BB = 256
GRID = B // BB
PR = T // PK
BBR = BB * PR
SET_P = SET // PK


def _pack4(m):
  return jnp.concatenate([m, m, m, m], axis=0)


def _tile4(m):
  return jnp.concatenate([m, m, m, m], axis=1)


def _tc_body(ent_ref, rel_ref, item_ref, w1_ref, w2_ref, w3_ref, out_ref):
  gi = jax.lax.broadcasted_iota(jnp.int32, (128, 128), 0) // DIM
  gj = jax.lax.broadcasted_iota(jnp.int32, (128, 128), 1) // DIM
  gmask = (gi == gj).astype(f32)
  w1ap = _tile4(_pack4(w1_ref[0:DIM, :])) * gmask
  w1bp = _tile4(_pack4(w1_ref[DIM:2 * DIM, :])) * gmask
  w2p = _tile4(_pack4(w2_ref[...])) * gmask
  m3 = jnp.broadcast_to(_pack4(w3_ref[...]), (128, 128)) * gmask

  def attend(hp, rp, tp):
    x = jax.nn.relu(jnp.dot(hp, w1ap, preferred_element_type=f32)
                    + jnp.dot(rp, w1bp, preferred_element_type=f32))
    x = jax.nn.relu(jnp.dot(x, w2p, preferred_element_type=f32))
    sb = jnp.dot(x, m3, preferred_element_type=f32)
    pe = jnp.exp(jax.nn.sigmoid(sb)).reshape(BB, PR, 128)
    denom = pe.sum(axis=1).sum(axis=-1, keepdims=True) * (1.0 / T)
    att = pe / denom[:, :, None]
    tpf = tp.astype(f32).reshape(BB, PR, 128)
    return (att * tpf).sum(axis=1)

  u0p = attend(ent_ref[0], rel_ref[0], ent_ref[1])
  u1p = attend(ent_ref[2], rel_ref[1], ent_ref[3])
  i0p = attend(ent_ref[4], rel_ref[2], ent_ref[5])
  i1p = attend(ent_ref[6], rel_ref[3], ent_ref[7])
  uorg_p = ent_ref[0].astype(f32).reshape(BB, PR, 128).sum(axis=1)
  iorg4 = _tile4(item_ref[...])

  fl = jax.lax.broadcasted_iota(jnp.int32, (128, DIM), 0) % DIM
  fd = jax.lax.broadcasted_iota(jnp.int32, (128, DIM), 1)
  fmat = (fl == fd).astype(f32)
  vl0 = _tile4(jnp.dot(i0p, fmat, preferred_element_type=f32))
  vl1 = _tile4(jnp.dot(i1p, fmat, preferred_element_type=f32))

  pre = ((uorg_p * iorg4).sum(axis=-1) * (1.0 / T)
         + (u0p * vl0).sum(axis=-1) + (u1p * vl1).sum(axis=-1))
  out_ref[...] = jax.nn.sigmoid(pre)


def _tc_dense(ent3, rel3h, item_rows, W1, W2, W3, goff):
  return pl.pallas_call(
      _tc_body,
      grid=(GRID // 2,),
      in_specs=[
          pl.BlockSpec((ENT_SETS, BBR, 128), lambda g: (0, g + goff, 0)),
          pl.BlockSpec((REL_SETS, BBR, 128), lambda g: (0, g, 0)),
          pl.BlockSpec((BB, DIM), lambda g: (g + goff, 0)),
          pl.BlockSpec((2 * DIM, DIM), lambda g: (0, 0)),
          pl.BlockSpec((DIM, DIM), lambda g: (0, 0)),
          pl.BlockSpec((DIM, 1), lambda g: (0, 0)),
      ],
      out_specs=pl.BlockSpec((BB,), lambda g: (g,)),
      out_shape=jax.ShapeDtypeStruct((B // 2,), f32),
  )(ent3, rel3h, item_rows, W1, W2, W3)


def kernel(users, items, user_triple_set, item_triple_set,
           entity_emb, relation_emb, W1, W2, W3):
  uts = user_triple_set
  its = item_triple_set
  ent_idx = jnp.concatenate([
      uts[0, 0].reshape(-1), uts[2, 0].reshape(-1),
      uts[0, 1].reshape(-1), uts[2, 1].reshape(-1),
      its[0, 0].reshape(-1), its[2, 0].reshape(-1),
      its[0, 1].reshape(-1), its[2, 1].reshape(-1),
  ])
  hb = B // 2
  rel_idx_h = [jnp.concatenate([
      uts[1, 0, h * hb:(h + 1) * hb].reshape(-1),
      uts[1, 1, h * hb:(h + 1) * hb].reshape(-1),
      its[1, 0, h * hb:(h + 1) * hb].reshape(-1),
      its[1, 1, h * hb:(h + 1) * hb].reshape(-1),
  ]) for h in range(2)]
  def _remap(i):
    return jnp.where(i < TAIL_BASE,
                     (i % REGION) * PK + i // REGION,
                     (i - TAIL_BASE + TAIL_R) * PK + (PK - 1))

  ent_idx = _remap(ent_idx)
  rel_idx_h = [_remap(r) for r in rel_idx_h]
  item_idx = _remap(items)
  ent_lin = _tc_table_prep(entity_emb.T, entity_emb[TAIL_BASE:])
  ent_rows, item_rows = _sc_gather_ent(
      ent_lin.reshape(N_PAD, DIM), ent_idx, item_idx)
  rel_lin = _tc_table_prep(relation_emb.T, relation_emb[TAIL_BASE:])
  rel_h = [_sc_gather_rel(rel_lin.reshape(N_PAD, DIM), r)
           for r in rel_idx_h]
  ent3 = ent_rows.reshape(ENT_SETS, SET_P, 128)
  halves = [
      _tc_dense(ent3, rel_h[h].reshape(REL_SETS, SET_P // 2, 128),
                item_rows, W1, W2, W3, h * (GRID // 2))
      for h in range(2)
  ]
  scores = jnp.concatenate(halves)
  return scores, 0.0, 0.0, 0.0
